# initial kernel scaffold (unmeasured)
import jax
import jax.numpy as jnp
from jax import lax
from jax.experimental import pallas as pl
from jax.experimental.pallas import tpu as pltpu


def kernel(
    x,
):
    def body(*refs):
        pass

    out_shape = jax.ShapeDtypeStruct(..., jnp.float32)
    return pl.pallas_call(body, out_shape=out_shape)(...)


# baseline (device time: 228949 ns/iter reference)
import jax
import jax.numpy as jnp
from jax import lax
from jax.experimental import pallas as pl
from jax.experimental.pallas import tpu as pltpu

N_X = 2


def kernel(x):
    m, n = x.shape
    xb = x.astype(jnp.bfloat16)

    def body(x_ref, out_ref, send_sem, recv_sem):
        my_x = lax.axis_index("x")
        my_y = lax.axis_index("y")
        my_z = lax.axis_index("z")
        peer = (1 - my_x, my_y, my_z)

        barrier_sem = pltpu.get_barrier_semaphore()
        pl.semaphore_signal(
            barrier_sem, inc=1, device_id=peer,
            device_id_type=pl.DeviceIdType.MESH,
        )
        pl.semaphore_wait(barrier_sem, 1)

        rdma = pltpu.make_async_remote_copy(
            src_ref=x_ref,
            dst_ref=out_ref.at[pl.ds(my_x * m, m), :],
            send_sem=send_sem,
            recv_sem=recv_sem,
            device_id=peer,
            device_id_type=pl.DeviceIdType.MESH,
        )
        rdma.start()

        out_ref[pl.ds(my_x * m, m), :] = x_ref[...]

        rdma.wait()

    return pl.pallas_call(
        body,
        out_shape=jax.ShapeDtypeStruct((N_X * m, n), jnp.bfloat16),
        in_specs=[pl.BlockSpec(memory_space=pltpu.VMEM)],
        out_specs=pl.BlockSpec(memory_space=pltpu.VMEM),
        scratch_shapes=[
            pltpu.SemaphoreType.DMA,
            pltpu.SemaphoreType.DMA,
        ],
        compiler_params=pltpu.CompilerParams(collective_id=0),
    )(xb)


# device time: 147574 ns/iter; 1.5514x vs baseline; 1.5514x over previous
import jax
import jax.numpy as jnp
from jax import lax
from jax.experimental import pallas as pl
from jax.experimental.pallas import tpu as pltpu

N_X = 2
K = 16


def kernel(x):
    m, n = x.shape
    half = m // 2
    c = half // K
    xb = x.astype(jnp.bfloat16)

    def body(x_ref, out_ref, x_send, x_recv, y_send, y_recv, loc_sem):
        my_x = lax.axis_index("x")
        my_y = lax.axis_index("y")
        my_z = lax.axis_index("z")
        x_peer = (1 - my_x, my_y, my_z)
        y_peer = (my_x, 1 - my_y, my_z)

        miss = (1 - my_x) * m
        mine = my_y * half

        barrier_sem = pltpu.get_barrier_semaphore()
        for nbr in [x_peer, y_peer]:
            pl.semaphore_signal(
                barrier_sem, inc=1, device_id=nbr,
                device_id_type=pl.DeviceIdType.MESH,
            )
        pl.semaphore_wait(barrier_sem, 2)

        x_rdmas = []
        for k in range(K):
            rdma = pltpu.make_async_remote_copy(
                src_ref=x_ref.at[pl.ds(mine + k * c, c), :],
                dst_ref=out_ref.at[pl.ds(my_x * m + mine + k * c, c), :],
                send_sem=x_send.at[k],
                recv_sem=x_recv.at[k],
                device_id=x_peer,
                device_id_type=pl.DeviceIdType.MESH,
            )
            rdma.start()
            x_rdmas.append(rdma)

        own = pltpu.make_async_copy(
            x_ref, out_ref.at[pl.ds(my_x * m, m), :], loc_sem,
        )
        own.start()

        y_rdmas = []
        for k in range(K):
            x_rdmas[k].wait_recv()
            rdma = pltpu.make_async_remote_copy(
                src_ref=out_ref.at[pl.ds(miss + mine + k * c, c), :],
                dst_ref=out_ref.at[pl.ds(miss + mine + k * c, c), :],
                send_sem=y_send.at[k],
                recv_sem=y_recv.at[k],
                device_id=y_peer,
                device_id_type=pl.DeviceIdType.MESH,
            )
            rdma.start()
            y_rdmas.append(rdma)

        for k in range(K):
            y_rdmas[k].wait_recv()
        for k in range(K):
            x_rdmas[k].wait_send()
            y_rdmas[k].wait_send()
        own.wait()

    return pl.pallas_call(
        body,
        out_shape=jax.ShapeDtypeStruct((N_X * m, n), jnp.bfloat16),
        in_specs=[pl.BlockSpec(memory_space=pltpu.VMEM)],
        out_specs=pl.BlockSpec(memory_space=pltpu.VMEM),
        scratch_shapes=[
            pltpu.SemaphoreType.DMA((K,)),
            pltpu.SemaphoreType.DMA((K,)),
            pltpu.SemaphoreType.DMA((K,)),
            pltpu.SemaphoreType.DMA((K,)),
            pltpu.SemaphoreType.DMA,
        ],
        compiler_params=pltpu.CompilerParams(collective_id=0),
    )(xb)


# device time: 140741 ns/iter; 1.6267x vs baseline; 1.0486x over previous
import jax
import jax.numpy as jnp
from jax import lax
from jax.experimental import pallas as pl
from jax.experimental.pallas import tpu as pltpu

N_X = 2
K = 16


def kernel(x):
    m, n = x.shape
    half = m // 2
    c = half // K
    n_chunks = m // c

    def body(x_hbm, out_hbm, own_bf16, stag, stag_sem,
             x_send, x_recv, y_send, y_recv, own_sem):
        my_x = lax.axis_index("x")
        my_y = lax.axis_index("y")
        my_z = lax.axis_index("z")
        x_peer = (1 - my_x, my_y, my_z)
        y_peer = (my_x, 1 - my_y, my_z)

        miss = (1 - my_x) * m
        mine = my_y * half

        barrier_sem = pltpu.get_barrier_semaphore()
        for nbr in [x_peer, y_peer]:
            pl.semaphore_signal(
                barrier_sem, inc=1, device_id=nbr,
                device_id_type=pl.DeviceIdType.MESH,
            )
        pl.semaphore_wait(barrier_sem, 2)

        other = (1 - my_y) * half
        offs = [mine + k * c for k in range(K)] + [other + k * c for k in range(K)]

        def stage_start(off, slot):
            cp = pltpu.make_async_copy(
                x_hbm.at[pl.ds(off, c), :], stag.at[slot], stag_sem.at[slot],
            )
            cp.start()
            return cp

        x_rdmas = []
        cps = {0: stage_start(offs[0], 0)}
        for i in range(n_chunks):
            slot = i % 2
            if i + 1 < n_chunks:
                cps[(i + 1) % 2] = stage_start(offs[i + 1], (i + 1) % 2)
            cps[slot].wait()
            own_bf16[pl.ds(offs[i], c), :] = stag[slot, :, :].astype(jnp.bfloat16)
            if i < K:
                k = i
                rdma = pltpu.make_async_remote_copy(
                    src_ref=own_bf16.at[pl.ds(mine + k * c, c), :],
                    dst_ref=out_hbm.at[pl.ds(my_x * m + mine + k * c, c), :],
                    send_sem=x_send.at[k],
                    recv_sem=x_recv.at[k],
                    device_id=x_peer,
                    device_id_type=pl.DeviceIdType.MESH,
                )
                rdma.start()
                x_rdmas.append(rdma)

        own = pltpu.make_async_copy(
            own_bf16, out_hbm.at[pl.ds(my_x * m, m), :], own_sem,
        )
        own.start()

        y_rdmas = []
        for k in range(K):
            x_rdmas[k].wait_recv()
            rdma = pltpu.make_async_remote_copy(
                src_ref=out_hbm.at[pl.ds(miss + mine + k * c, c), :],
                dst_ref=out_hbm.at[pl.ds(miss + mine + k * c, c), :],
                send_sem=y_send.at[k],
                recv_sem=y_recv.at[k],
                device_id=y_peer,
                device_id_type=pl.DeviceIdType.MESH,
            )
            rdma.start()
            y_rdmas.append(rdma)

        for k in range(K):
            y_rdmas[k].wait_recv()
        for k in range(K):
            x_rdmas[k].wait_send()
            y_rdmas[k].wait_send()
        own.wait()

    return pl.pallas_call(
        body,
        out_shape=jax.ShapeDtypeStruct((N_X * m, n), jnp.bfloat16),
        in_specs=[pl.BlockSpec(memory_space=pl.ANY)],
        out_specs=pl.BlockSpec(memory_space=pl.ANY),
        scratch_shapes=[
            pltpu.VMEM((m, n), jnp.bfloat16),
            pltpu.VMEM((2, c, n), jnp.float32),
            pltpu.SemaphoreType.DMA((2,)),
            pltpu.SemaphoreType.DMA((K,)),
            pltpu.SemaphoreType.DMA((K,)),
            pltpu.SemaphoreType.DMA((K,)),
            pltpu.SemaphoreType.DMA((K,)),
            pltpu.SemaphoreType.DMA,
        ],
        compiler_params=pltpu.CompilerParams(collective_id=0),
    )(x)


# device time: 130135 ns/iter; 1.7593x vs baseline; 1.0815x over previous
import jax
import jax.numpy as jnp
from jax import lax
from jax.experimental import pallas as pl
from jax.experimental.pallas import tpu as pltpu

N_X = 2
K = 16


def kernel(x):
    m, n = x.shape
    half = m // 2
    c = half // K
    n_chunks = m // c

    def body(x_hbm, out_hbm, own_bf16, stag, stag_sem,
             x_send, x_recv, y_send, y_recv, own_sem):
        my_x = lax.axis_index("x")
        my_y = lax.axis_index("y")
        my_z = lax.axis_index("z")
        x_peer = (1 - my_x, my_y, my_z)
        y_peer = (my_x, 1 - my_y, my_z)

        miss = (1 - my_x) * m
        mine = my_y * half

        barrier_sem = pltpu.get_barrier_semaphore()
        for nbr in [x_peer, y_peer]:
            pl.semaphore_signal(
                barrier_sem, inc=1, device_id=nbr,
                device_id_type=pl.DeviceIdType.MESH,
            )
        pl.semaphore_wait(barrier_sem, 2)

        other = (1 - my_y) * half

        def stage_start(off, slot):
            cp = pltpu.make_async_copy(
                x_hbm.at[pl.ds(off, c), :], stag.at[slot], stag_sem.at[slot],
            )
            cp.start()
            return cp

        x_rdmas = []
        cps = {0: stage_start(mine, 0)}
        for k in range(K):
            slot = k % 2
            if k + 1 < K:
                cps[(k + 1) % 2] = stage_start(mine + (k + 1) * c, (k + 1) % 2)
            cps[slot].wait()
            own_bf16[pl.ds(mine + k * c, c), :] = (
                stag[slot, :, :].astype(jnp.bfloat16))
            rdma = pltpu.make_async_remote_copy(
                src_ref=own_bf16.at[pl.ds(mine + k * c, c), :],
                dst_ref=out_hbm.at[pl.ds(my_x * m + mine + k * c, c), :],
                send_sem=x_send.at[k],
                recv_sem=x_recv.at[k],
                device_id=x_peer,
                device_id_type=pl.DeviceIdType.MESH,
            )
            rdma.start()
            x_rdmas.append(rdma)

        own_a = pltpu.make_async_copy(
            own_bf16.at[pl.ds(mine, half), :],
            out_hbm.at[pl.ds(my_x * m + mine, half), :],
            own_sem.at[0],
        )
        own_a.start()

        y_rdmas = []
        cps = {0: stage_start(other, 0)}
        for k in range(K):
            x_rdmas[k].wait_recv()
            rdma = pltpu.make_async_remote_copy(
                src_ref=out_hbm.at[pl.ds(miss + mine + k * c, c), :],
                dst_ref=out_hbm.at[pl.ds(miss + mine + k * c, c), :],
                send_sem=y_send.at[k],
                recv_sem=y_recv.at[k],
                device_id=y_peer,
                device_id_type=pl.DeviceIdType.MESH,
            )
            rdma.start()
            y_rdmas.append(rdma)
            slot = k % 2
            if k + 1 < K:
                cps[(k + 1) % 2] = stage_start(other + (k + 1) * c, (k + 1) % 2)
            cps[slot].wait()
            own_bf16[pl.ds(other + k * c, c), :] = (
                stag[slot, :, :].astype(jnp.bfloat16))

        own_b = pltpu.make_async_copy(
            own_bf16.at[pl.ds(other, half), :],
            out_hbm.at[pl.ds(my_x * m + other, half), :],
            own_sem.at[1],
        )
        own_b.start()

        for k in range(K):
            y_rdmas[k].wait_recv()
        for k in range(K):
            x_rdmas[k].wait_send()
            y_rdmas[k].wait_send()
        own_a.wait()
        own_b.wait()

    return pl.pallas_call(
        body,
        out_shape=jax.ShapeDtypeStruct((N_X * m, n), jnp.bfloat16),
        in_specs=[pl.BlockSpec(memory_space=pl.ANY)],
        out_specs=pl.BlockSpec(memory_space=pl.ANY),
        scratch_shapes=[
            pltpu.VMEM((m, n), jnp.bfloat16),
            pltpu.VMEM((2, c, n), jnp.float32),
            pltpu.SemaphoreType.DMA((2,)),
            pltpu.SemaphoreType.DMA((K,)),
            pltpu.SemaphoreType.DMA((K,)),
            pltpu.SemaphoreType.DMA((K,)),
            pltpu.SemaphoreType.DMA((K,)),
            pltpu.SemaphoreType.DMA((2,)),
        ],
        compiler_params=pltpu.CompilerParams(collective_id=0),
    )(x)
